# baseline (speedup 1.0000x reference)
import functools

import jax
import jax.numpy as jnp
from jax import lax
from jax.experimental import pallas as pl
from jax.experimental.pallas import tpu as pltpu
from jax.experimental.pallas import tpu_sc as plsc

N = 10000
D = 128
E = 320000
NC = 2
NS = 16
L = 16
NPAD = 10240
RPT = NPAD // NS
C = 64
EPAD = 327680
NCHUNK = EPAD // NS // C
NIDX = 8
NROW = 4


def _seg_body(with_deg, tbl_u, tbl_m, edges_r, edges_v, zeros_hbm,
              *refs):
    if with_deg:
        (agg_out, deg_out, *rest) = refs
    else:
        (agg_out, *rest) = refs
    idx = rest[:NIDX]
    rows = rest[NIDX:NIDX + NROW]
    zrow_v = rest[NIDX + NROW]
    rest = rest[NIDX + NROW + 1:]
    if with_deg:
        ones_v, zdeg_v, acc_sh, deg_sh = rest[:4]
        rest = rest[4:]
    else:
        acc_sh = rest[0]
        rest = rest[1:]
    isem = rest[:NIDX]
    gsem = rest[NIDX:NIDX + NROW]
    ssem = rest[NIDX + NROW:NIDX + 2 * NROW]
    dsem = rest[NIDX + 2 * NROW:]
    c = lax.axis_index("c")
    s = lax.axis_index("s")

    pltpu.sync_copy(zeros_hbm, zrow_v)
    for k in range(RPT // C):
        pltpu.sync_copy(zrow_v, acc_sh.at[pl.ds(s * RPT + k * C, C)])
    if with_deg:
        for j in range(C // L):
            ones_v[pl.ds(j * L, L)] = jnp.ones((L,), jnp.float32)

        def zd(j, carry):
            zdeg_v[pl.ds(j * L, L)] = jnp.zeros((L,), jnp.float32)
            return carry
        lax.fori_loop(0, RPT // L, zd, 0)
        pltpu.sync_copy(zdeg_v, deg_sh.at[pl.ds(s * RPT, RPT)])
    plsc.subcore_barrier()

    def run(tbl, edges, slot):
        base = s * NCHUNK

        def issue_idx(i, q):
            pltpu.async_copy(edges.at[base + i], idx[q], isem[q])

        def wait_idx(i, q):
            pltpu.make_async_copy(edges.at[base + i], idx[q], isem[q]).wait()

        def issue_gather(q, r):
            pltpu.async_copy(tbl.at[idx[q].at[0]], rows[r], gsem[r])

        def wait_gather(q, r):
            pltpu.make_async_copy(tbl.at[idx[q].at[0]], rows[r],
                                  gsem[r]).wait()

        def issue_scatter(q, r):
            pltpu.async_copy(rows[r], acc_sh.at[idx[q].at[1]], ssem[r],
                             add=True)
            if with_deg:
                pltpu.async_copy(ones_v, deg_sh.at[idx[q].at[1]], dsem[r],
                                 add=True)

        def wait_scatter(q, r):
            pltpu.make_async_copy(rows[r], acc_sh.at[idx[q].at[1]],
                                  ssem[r]).wait()
            if with_deg:
                pltpu.make_async_copy(ones_v, deg_sh.at[idx[q].at[1]],
                                      dsem[r]).wait()

        issue_idx(0, 0)
        issue_idx(1, 1)
        wait_idx(0, 0)
        issue_gather(0, 0)

        def group(g, carry):
            for k in range(NIDX):
                i = g * NIDX + k

                @pl.when(i + 2 < NCHUNK)
                def _():
                    issue_idx(i + 2, (k + 2) % NIDX)

                @pl.when(i >= NROW - 1)
                def _():
                    wait_scatter((k - NROW + 1) % NIDX, (k - NROW + 1) % NROW)

                @pl.when(i + 1 < NCHUNK)
                def _():
                    wait_idx(i + 1, (k + 1) % NIDX)
                    issue_gather((k + 1) % NIDX, (k + 1) % NROW)

                wait_gather(k, k % NROW)
                issue_scatter(k, k % NROW)
            return carry

        lax.fori_loop(0, NCHUNK // NIDX, group, 0)
        for j in range(NCHUNK - NROW + 1, NCHUNK):
            wait_scatter(j % NIDX, j % NROW)
        plsc.subcore_barrier()
        pltpu.sync_copy(acc_sh.at[pl.ds(s * RPT, RPT)],
                        agg_out.at[slot, pl.ds(s * RPT, RPT)])
        if with_deg:
            pltpu.sync_copy(deg_sh.at[pl.ds(s * RPT, RPT)],
                            deg_out.at[slot, 0, pl.ds(s * RPT, RPT)])

    @pl.when(c == 0)
    def _():
        run(tbl_u, edges_r, 0)

    @pl.when(c == 1)
    def _():
        run(tbl_m, edges_v, 1)


@functools.cache
def _seg(with_deg):
    out_type = [jax.ShapeDtypeStruct((2, NPAD, D), jnp.bfloat16)]
    if with_deg:
        out_type.append(jax.ShapeDtypeStruct((2, 1, NPAD), jnp.float32))
    scratch = [pltpu.VMEM((2, C), jnp.int32) for _ in range(NIDX)]
    scratch += [pltpu.VMEM((C, D), jnp.bfloat16) for _ in range(NROW)]
    scratch.append(pltpu.VMEM((C, D), jnp.bfloat16))
    if with_deg:
        scratch += [
            pltpu.VMEM((C,), jnp.float32),
            pltpu.VMEM((RPT,), jnp.float32),
        ]
    scratch.append(pltpu.VMEM_SHARED((NPAD, D), jnp.bfloat16))
    if with_deg:
        scratch.append(pltpu.VMEM_SHARED((NPAD,), jnp.float32))
    nsem = NIDX + 2 * NROW + (NROW if with_deg else 0)
    scratch += [pltpu.SemaphoreType.DMA] * nsem
    return pl.kernel(
        functools.partial(_seg_body, with_deg),
        mesh=plsc.VectorSubcoreMesh(core_axis_name="c", subcore_axis_name="s"),
        out_type=out_type,
        scratch_types=scratch,
        compiler_params=pltpu.CompilerParams(use_tc_tiling_on_sc=False),
    )


def _dense_body(agg_ref, deg_ref, x_ref, wl_ref, wr_ref, b_ref, o_ref, *, relu):
    deg = deg_ref[0, 0]
    inv = 1.0 / jnp.maximum(deg, 1.0)
    agg = agg_ref[0].astype(jnp.float32) * inv[:, None]
    out = lax.dot_general(agg, wl_ref[0], (((1,), (1,)), ((), ())),
                          preferred_element_type=jnp.float32)
    out = out + lax.dot_general(x_ref[0], wr_ref[0], (((1,), (1,)), ((), ())),
                                preferred_element_type=jnp.float32)
    out = out + b_ref[0]
    o_ref[0] = jnp.maximum(out, 0.0) if relu else out


def _dense(agg, deg, xs, wl, wr, b, relu):
    B = 1024
    return pl.pallas_call(
        functools.partial(_dense_body, relu=relu),
        grid=(2, NPAD // B),
        in_specs=[
            pl.BlockSpec((1, B, D), lambda t, r: (t, r, 0)),
            pl.BlockSpec((1, 1, B), lambda t, r: (t, 0, r)),
            pl.BlockSpec((1, B, D), lambda t, r: (t, r, 0)),
            pl.BlockSpec((1, D, D), lambda t, r: (t, 0, 0)),
            pl.BlockSpec((1, D, D), lambda t, r: (t, 0, 0)),
            pl.BlockSpec((1, 1, D), lambda t, r: (t, 0, 0)),
        ],
        out_specs=pl.BlockSpec((1, B, D), lambda t, r: (t, r, 0)),
        out_shape=jax.ShapeDtypeStruct((2, NPAD, D), jnp.float32),
    )(agg, deg, xs, wl, wr, b)


def _pack_edges(edge_index):
    src = edge_index[0].astype(jnp.int32)
    dst = edge_index[1].astype(jnp.int32)
    pad = EPAD - E
    src = jnp.concatenate([src, jnp.zeros((pad,), jnp.int32)])
    dst = jnp.concatenate([dst, jnp.full((pad,), NPAD - 1, jnp.int32)])
    return jnp.stack([src.reshape(-1, C), dst.reshape(-1, C)], axis=1)


def kernel(x_user, x_movie, W_l1_um, b1_um, W_r1_um, W_l1_mu, b1_mu, W_r1_mu,
           W_l2_um, b2_um, W_r2_um, W_l2_mu, b2_mu, W_r2_mu,
           edge_index_rates, edge_index_rev):
    edges_r = _pack_edges(edge_index_rates)
    edges_v = _pack_edges(edge_index_rev)
    zeros = jnp.zeros((C, D), jnp.bfloat16)

    agg1, deg = _seg(True)(x_user.astype(jnp.bfloat16),
                           x_movie.astype(jnp.bfloat16),
                           edges_r, edges_v, zeros)
    wl1 = jnp.stack([W_l1_um, W_l1_mu])
    wr1 = jnp.stack([W_r1_um, W_r1_mu])
    b1 = jnp.stack([b1_um, b1_mu])[:, None, :]
    xs1 = jnp.zeros((2, NPAD, D), jnp.float32)
    xs1 = xs1.at[0, :N].set(x_movie).at[1, :N].set(x_user)
    h = _dense(agg1, deg, xs1, wl1, wr1, b1, relu=True)

    (agg2,) = _seg(False)(h[1, :N].astype(jnp.bfloat16),
                          h[0, :N].astype(jnp.bfloat16), edges_r, edges_v,
                          zeros)
    wl2 = jnp.stack([W_l2_um, W_l2_mu])
    wr2 = jnp.stack([W_r2_um, W_r2_mu])
    b2 = jnp.stack([b2_um, b2_mu])[:, None, :]
    o = _dense(agg2, deg, h, wl2, wr2, b2, relu=False)
    return (o[1, :N], o[0, :N])


# speedup vs baseline: 1.0719x; 1.0719x over previous
import functools

import jax
import jax.numpy as jnp
from jax import lax
from jax.experimental import pallas as pl
from jax.experimental.pallas import tpu as pltpu
from jax.experimental.pallas import tpu_sc as plsc

N = 10000
D = 128
E = 320000
NC = 2
NS = 16
L = 16
NPAD = 10240
RPT = NPAD // NS
C = 128
EPAD = 327680
NCHUNK = EPAD // NS // C
NIDX = 8
NROW = 4


def _seg_body(with_deg, tbl_u, tbl_m, edges_r, edges_v, zeros_hbm,
              *refs):
    if with_deg:
        (agg_out, deg_out, *rest) = refs
    else:
        (agg_out, *rest) = refs
    idx = rest[:NIDX]
    rows = rest[NIDX:NIDX + NROW]
    zrow_v = rest[NIDX + NROW]
    rest = rest[NIDX + NROW + 1:]
    if with_deg:
        ones_v, zdeg_v, acc_sh, deg_sh = rest[:4]
        rest = rest[4:]
    else:
        acc_sh = rest[0]
        rest = rest[1:]
    isem = rest[:NIDX]
    gsem = rest[NIDX:NIDX + NROW]
    ssem = rest[NIDX + NROW:NIDX + 2 * NROW]
    dsem = rest[NIDX + 2 * NROW:]
    c = lax.axis_index("c")
    s = lax.axis_index("s")

    pltpu.sync_copy(zeros_hbm, zrow_v)
    for k in range(RPT // C):
        pltpu.sync_copy(zrow_v, acc_sh.at[pl.ds(s * RPT + k * C, C)])
    if with_deg:
        for j in range(C // L):
            ones_v[pl.ds(j * L, L)] = jnp.ones((L,), jnp.float32)

        def zd(j, carry):
            zdeg_v[pl.ds(j * L, L)] = jnp.zeros((L,), jnp.float32)
            return carry
        lax.fori_loop(0, RPT // L, zd, 0)
        pltpu.sync_copy(zdeg_v, deg_sh.at[pl.ds(s * RPT, RPT)])
    plsc.subcore_barrier()

    def run(tbl, edges, slot):
        base = s * NCHUNK

        def issue_idx(i, q):
            pltpu.async_copy(edges.at[base + i], idx[q], isem[q])

        def wait_idx(i, q):
            pltpu.make_async_copy(edges.at[base + i], idx[q], isem[q]).wait()

        def issue_gather(q, r):
            pltpu.async_copy(tbl.at[idx[q].at[0]], rows[r], gsem[r])

        def wait_gather(q, r):
            pltpu.make_async_copy(tbl.at[idx[q].at[0]], rows[r],
                                  gsem[r]).wait()

        def issue_scatter(q, r):
            pltpu.async_copy(rows[r], acc_sh.at[idx[q].at[1]], ssem[r],
                             add=True)
            if with_deg:
                pltpu.async_copy(ones_v, deg_sh.at[idx[q].at[1]], dsem[r],
                                 add=True)

        def wait_scatter(q, r):
            pltpu.make_async_copy(rows[r], acc_sh.at[idx[q].at[1]],
                                  ssem[r]).wait()
            if with_deg:
                pltpu.make_async_copy(ones_v, deg_sh.at[idx[q].at[1]],
                                      dsem[r]).wait()

        issue_idx(0, 0)
        issue_idx(1, 1)
        wait_idx(0, 0)
        issue_gather(0, 0)

        def group(g, carry):
            for k in range(NIDX):
                i = g * NIDX + k

                @pl.when(i + 2 < NCHUNK)
                def _():
                    issue_idx(i + 2, (k + 2) % NIDX)

                @pl.when(i >= NROW - 1)
                def _():
                    wait_scatter((k - NROW + 1) % NIDX, (k - NROW + 1) % NROW)

                @pl.when(i + 1 < NCHUNK)
                def _():
                    wait_idx(i + 1, (k + 1) % NIDX)
                    issue_gather((k + 1) % NIDX, (k + 1) % NROW)

                wait_gather(k, k % NROW)
                issue_scatter(k, k % NROW)
            return carry

        lax.fori_loop(0, NCHUNK // NIDX, group, 0)
        for j in range(NCHUNK - NROW + 1, NCHUNK):
            wait_scatter(j % NIDX, j % NROW)
        plsc.subcore_barrier()
        pltpu.sync_copy(acc_sh.at[pl.ds(s * RPT, RPT)],
                        agg_out.at[slot, pl.ds(s * RPT, RPT)])
        if with_deg:
            pltpu.sync_copy(deg_sh.at[pl.ds(s * RPT, RPT)],
                            deg_out.at[slot, 0, pl.ds(s * RPT, RPT)])

    @pl.when(c == 0)
    def _():
        run(tbl_u, edges_r, 0)

    @pl.when(c == 1)
    def _():
        run(tbl_m, edges_v, 1)


@functools.cache
def _seg(with_deg):
    out_type = [jax.ShapeDtypeStruct((2, NPAD, D), jnp.bfloat16)]
    if with_deg:
        out_type.append(jax.ShapeDtypeStruct((2, 1, NPAD), jnp.float32))
    scratch = [pltpu.VMEM((2, C), jnp.int32) for _ in range(NIDX)]
    scratch += [pltpu.VMEM((C, D), jnp.bfloat16) for _ in range(NROW)]
    scratch.append(pltpu.VMEM((C, D), jnp.bfloat16))
    if with_deg:
        scratch += [
            pltpu.VMEM((C,), jnp.float32),
            pltpu.VMEM((RPT,), jnp.float32),
        ]
    scratch.append(pltpu.VMEM_SHARED((NPAD, D), jnp.bfloat16))
    if with_deg:
        scratch.append(pltpu.VMEM_SHARED((NPAD,), jnp.float32))
    nsem = NIDX + 2 * NROW + (NROW if with_deg else 0)
    scratch += [pltpu.SemaphoreType.DMA] * nsem
    return pl.kernel(
        functools.partial(_seg_body, with_deg),
        mesh=plsc.VectorSubcoreMesh(core_axis_name="c", subcore_axis_name="s"),
        out_type=out_type,
        scratch_types=scratch,
        compiler_params=pltpu.CompilerParams(use_tc_tiling_on_sc=False),
    )


def _dense_body(agg_ref, deg_ref, x_ref, wl_ref, wr_ref, b_ref, o_ref, *, relu):
    deg = deg_ref[0, 0]
    inv = 1.0 / jnp.maximum(deg, 1.0)
    agg = agg_ref[0].astype(jnp.float32) * inv[:, None]
    out = lax.dot_general(agg, wl_ref[0], (((1,), (1,)), ((), ())),
                          preferred_element_type=jnp.float32)
    out = out + lax.dot_general(x_ref[0], wr_ref[0], (((1,), (1,)), ((), ())),
                                preferred_element_type=jnp.float32)
    out = out + b_ref[0]
    o_ref[0] = jnp.maximum(out, 0.0) if relu else out


def _dense(agg, deg, xs, wl, wr, b, relu):
    B = 1024
    return pl.pallas_call(
        functools.partial(_dense_body, relu=relu),
        grid=(2, NPAD // B),
        in_specs=[
            pl.BlockSpec((1, B, D), lambda t, r: (t, r, 0)),
            pl.BlockSpec((1, 1, B), lambda t, r: (t, 0, r)),
            pl.BlockSpec((1, B, D), lambda t, r: (t, r, 0)),
            pl.BlockSpec((1, D, D), lambda t, r: (t, 0, 0)),
            pl.BlockSpec((1, D, D), lambda t, r: (t, 0, 0)),
            pl.BlockSpec((1, 1, D), lambda t, r: (t, 0, 0)),
        ],
        out_specs=pl.BlockSpec((1, B, D), lambda t, r: (t, r, 0)),
        out_shape=jax.ShapeDtypeStruct((2, NPAD, D), jnp.float32),
    )(agg, deg, xs, wl, wr, b)


def _pack_edges(edge_index):
    src = edge_index[0].astype(jnp.int32)
    dst = edge_index[1].astype(jnp.int32)
    pad = EPAD - E
    src = jnp.concatenate([src, jnp.zeros((pad,), jnp.int32)])
    dst = jnp.concatenate([dst, jnp.full((pad,), NPAD - 1, jnp.int32)])
    return jnp.stack([src.reshape(-1, C), dst.reshape(-1, C)], axis=1)


def kernel(x_user, x_movie, W_l1_um, b1_um, W_r1_um, W_l1_mu, b1_mu, W_r1_mu,
           W_l2_um, b2_um, W_r2_um, W_l2_mu, b2_mu, W_r2_mu,
           edge_index_rates, edge_index_rev):
    edges_r = _pack_edges(edge_index_rates)
    edges_v = _pack_edges(edge_index_rev)
    zeros = jnp.zeros((C, D), jnp.bfloat16)

    agg1, deg = _seg(True)(x_user.astype(jnp.bfloat16),
                           x_movie.astype(jnp.bfloat16),
                           edges_r, edges_v, zeros)
    wl1 = jnp.stack([W_l1_um, W_l1_mu])
    wr1 = jnp.stack([W_r1_um, W_r1_mu])
    b1 = jnp.stack([b1_um, b1_mu])[:, None, :]
    xs1 = jnp.zeros((2, NPAD, D), jnp.float32)
    xs1 = xs1.at[0, :N].set(x_movie).at[1, :N].set(x_user)
    h = _dense(agg1, deg, xs1, wl1, wr1, b1, relu=True)

    (agg2,) = _seg(False)(h[1, :N].astype(jnp.bfloat16),
                          h[0, :N].astype(jnp.bfloat16), edges_r, edges_v,
                          zeros)
    wl2 = jnp.stack([W_l2_um, W_l2_mu])
    wr2 = jnp.stack([W_r2_um, W_r2_mu])
    b2 = jnp.stack([b2_um, b2_mu])[:, None, :]
    o = _dense(agg2, deg, h, wl2, wr2, b2, relu=False)
    return (o[1, :N], o[0, :N])
